# Initial kernel scaffold; baseline (speedup 1.0000x reference)
import jax
import jax.numpy as jnp
from jax.experimental import pallas as pl


def kernel(inputs):
    raise NotImplementedError("write your pallas kernel here")


# speedup vs baseline: 4.4800x; 4.4800x over previous
import functools

import jax
import jax.numpy as jnp
from jax import lax
from jax.experimental import pallas as pl
from jax.experimental.pallas import tpu as pltpu
from jax.experimental.pallas import tpu_sc as plsc

B_ROWS = 256000
M = 128
K = 500
NSEQ = B_ROWS // K
HALF = K // 2
NGROUP = (HALF - 1) // 3
NCHUNK = M // 16
L = 16
NW = 32
SEQ_PER_W = NSEQ // NW


def _sc_segment_mean(x):
    mesh = plsc.VectorSubcoreMesh(core_axis_name="c", subcore_axis_name="s")

    @functools.partial(
        pl.kernel,
        out_type=jax.ShapeDtypeStruct((NSEQ, 6 * M), jnp.float32),
        mesh=mesh,
        scratch_types=[
            pltpu.VMEM((K, M), jnp.float32),
            pltpu.VMEM((K, M), jnp.float32),
            pltpu.VMEM((6 * M,), jnp.float32),
            pltpu.SemaphoreType.DMA,
            pltpu.SemaphoreType.DMA,
        ],
    )
    def body(x_hbm, out_hbm, buf0, buf1, obuf, sem0, sem1):
        wid = lax.axis_index("s") * 2 + lax.axis_index("c")
        base = wid * SEQ_PER_W
        bufs = (buf0, buf1)
        sems = (sem0, sem1)

        copies = [pltpu.async_copy(x_hbm.at[base], buf0, sem0)]
        for i in range(SEQ_PER_W):
            seq = base + i
            if i + 1 < SEQ_PER_W:
                copies.append(
                    pltpu.async_copy(
                        x_hbm.at[seq + 1], bufs[(i + 1) % 2], sems[(i + 1) % 2]
                    )
                )
            copies[i].wait()
            buf = bufs[i % 2]

            for h in range(2):
                rbase = h * HALF

                def group_body(t, accs, rbase=rbase, buf=buf):
                    accs = list(accs)
                    row0 = rbase + 3 * t
                    for r in range(3):
                        for c in range(NCHUNK):
                            v = buf[row0 + r, pl.ds(c * L, L)]
                            accs[r * NCHUNK + c] = accs[r * NCHUNK + c] + v
                    return tuple(accs)

                init = tuple(jnp.zeros((L,), jnp.float32) for _ in range(3 * NCHUNK))
                accs = list(lax.fori_loop(0, NGROUP, group_body, init))
                for c in range(NCHUNK):
                    accs[c] = accs[c] + buf[rbase + HALF - 1, pl.ds(c * L, L)]
                for r in range(3):
                    scale = 1.0 / float(NGROUP + (1 if r == 0 else 0))
                    for c in range(NCHUNK):
                        obuf[pl.ds((3 * h + r) * M + c * L, L)] = (
                            accs[r * NCHUNK + c] * scale
                        )
            pltpu.sync_copy(obuf, out_hbm.at[seq])

    return body(x)


def kernel(inputs):
    x = inputs.reshape(NSEQ, K, M)
    return _sc_segment_mean(x)


# speedup vs baseline: 9.6379x; 2.1513x over previous
import functools

import jax
import jax.numpy as jnp
from jax import lax
from jax.experimental import pallas as pl
from jax.experimental.pallas import tpu as pltpu
from jax.experimental.pallas import tpu_sc as plsc

B_ROWS = 256000
M = 128
K = 500
NSEQ = B_ROWS // K
HALF = K // 2
NGROUP = (HALF - 1) // 3
NCHUNK = M // 16
L = 16
NW = 32
SEQ_PER_W = NSEQ // NW
SEQ_ELEMS = K * M
OUT_ELEMS = 6 * M


def _sc_segment_mean(x):
    mesh = plsc.VectorSubcoreMesh(core_axis_name="c", subcore_axis_name="s")

    @functools.partial(
        pl.kernel,
        out_type=jax.ShapeDtypeStruct((NSEQ * OUT_ELEMS,), jnp.float32),
        mesh=mesh,
        scratch_types=[
            pltpu.VMEM((SEQ_ELEMS,), jnp.float32),
            pltpu.VMEM((SEQ_ELEMS,), jnp.float32),
            pltpu.VMEM((OUT_ELEMS,), jnp.float32),
            pltpu.SemaphoreType.DMA,
            pltpu.SemaphoreType.DMA,
        ],
    )
    def body(x_hbm, out_hbm, buf0, buf1, obuf, sem0, sem1):
        wid = lax.axis_index("s") * 2 + lax.axis_index("c")
        base = wid * SEQ_PER_W
        bufs = (buf0, buf1)
        sems = (sem0, sem1)

        copies = [pltpu.async_copy(x_hbm.at[pl.ds(base * SEQ_ELEMS, SEQ_ELEMS)], buf0, sem0)]
        for i in range(SEQ_PER_W):
            seq = base + i
            if i + 1 < SEQ_PER_W:
                copies.append(
                    pltpu.async_copy(
                        x_hbm.at[pl.ds((seq + 1) * SEQ_ELEMS, SEQ_ELEMS)],
                        bufs[(i + 1) % 2],
                        sems[(i + 1) % 2],
                    )
                )
            copies[i].wait()
            buf = bufs[i % 2]

            for h in range(2):
                rbase = h * HALF

                def group_body(t, accs, rbase=rbase, buf=buf):
                    accs = list(accs)
                    row0 = rbase + 3 * t
                    for r in range(3):
                        for c in range(NCHUNK):
                            v = buf[pl.ds((row0 + r) * M + c * L, L)]
                            accs[r * NCHUNK + c] = accs[r * NCHUNK + c] + v
                    return tuple(accs)

                init = tuple(jnp.zeros((L,), jnp.float32) for _ in range(3 * NCHUNK))
                accs = list(lax.fori_loop(0, NGROUP, group_body, init))
                for c in range(NCHUNK):
                    accs[c] = accs[c] + buf[pl.ds((rbase + HALF - 1) * M + c * L, L)]
                for r in range(3):
                    scale = 1.0 / float(NGROUP + (1 if r == 0 else 0))
                    for c in range(NCHUNK):
                        obuf[pl.ds((3 * h + r) * M + c * L, L)] = (
                            accs[r * NCHUNK + c] * scale
                        )
            pltpu.sync_copy(obuf, out_hbm.at[pl.ds(seq * OUT_ELEMS, OUT_ELEMS)])

    return body(x)


def kernel(inputs):
    flat = _sc_segment_mean(inputs.reshape(-1))
    return flat.reshape(NSEQ, OUT_ELEMS)


# speedup vs baseline: 9.9560x; 1.0330x over previous
import functools

import jax
import jax.numpy as jnp
from jax import lax
from jax.experimental import pallas as pl
from jax.experimental.pallas import tpu as pltpu
from jax.experimental.pallas import tpu_sc as plsc

B_ROWS = 256000
M = 128
K = 500
NSEQ = B_ROWS // K
HALF = K // 2
NGROUP = (HALF - 1) // 3
UNROLL = 4
NLOOP = NGROUP // UNROLL
NTAIL = NGROUP - NLOOP * UNROLL
NCHUNK = M // 16
L = 16
NW = 32
SEQ_PER_W = NSEQ // NW
SEQ_ELEMS = K * M
OUT_ELEMS = 6 * M


def _acc_rows(buf, accs, row0, nrows):
    accs = list(accs)
    for r in range(nrows):
        for c in range(NCHUNK):
            v = buf[pl.ds((row0 + r) * M + c * L, L)]
            k = (r % 3) * NCHUNK + c
            accs[k] = accs[k] + v
    return accs


def _sc_segment_mean(x):
    mesh = plsc.VectorSubcoreMesh(core_axis_name="c", subcore_axis_name="s")

    @functools.partial(
        pl.kernel,
        out_type=jax.ShapeDtypeStruct((NSEQ, OUT_ELEMS), jnp.float32),
        mesh=mesh,
        scratch_types=[
            pltpu.VMEM((HALF * M,), jnp.float32),
            pltpu.VMEM((HALF * M,), jnp.float32),
            pltpu.VMEM((SEQ_PER_W, OUT_ELEMS), jnp.float32),
            pltpu.SemaphoreType.DMA,
            pltpu.SemaphoreType.DMA,
        ],
    )
    def body(x_hbm, out_hbm, buf0, buf1, obuf, sem0, sem1):
        wid = lax.axis_index("s") * 2 + lax.axis_index("c")
        base = wid * SEQ_PER_W

        def start(j, buf, sem):
            src = x_hbm.at[pl.ds(base * SEQ_ELEMS + j * (HALF * M), HALF * M)]
            pltpu.async_copy(src, buf, sem)

        def wait(buf, sem):
            pltpu.make_async_copy(
                x_hbm.at[pl.ds(0, HALF * M)], buf, sem
            ).wait()

        def compute(buf, i, h):
            def group_body(t, accs):
                return tuple(_acc_rows(buf, accs, 3 * UNROLL * t, 3 * UNROLL))

            init = tuple(jnp.zeros((L,), jnp.float32) for _ in range(3 * NCHUNK))
            accs = list(lax.fori_loop(0, NLOOP, group_body, init))
            accs = _acc_rows(buf, accs, 3 * NLOOP * UNROLL, 3 * NTAIL + 1)
            for r in range(3):
                scale = 1.0 / float(NGROUP + (1 if r == 0 else 0))
                for c in range(NCHUNK):
                    obuf[i, pl.ds((3 * h + r) * M + c * L, L)] = (
                        accs[r * NCHUNK + c] * scale
                    )

        start(0, buf0, sem0)

        def pair_body(t, carry):
            start(2 * t + 1, buf1, sem1)
            wait(buf0, sem0)
            compute(buf0, t, 0)

            @pl.when(t < SEQ_PER_W - 1)
            def _():
                start(2 * t + 2, buf0, sem0)

            wait(buf1, sem1)
            compute(buf1, t, 1)
            return carry

        lax.fori_loop(0, SEQ_PER_W, pair_body, 0)
        pltpu.sync_copy(obuf, out_hbm.at[pl.ds(base, SEQ_PER_W)])

    return body(x)


def kernel(inputs):
    return _sc_segment_mean(inputs.reshape(-1))


# speedup vs baseline: 10.8325x; 1.0880x over previous
import functools

import jax
import jax.numpy as jnp
from jax import lax
from jax.experimental import pallas as pl
from jax.experimental.pallas import tpu as pltpu
from jax.experimental.pallas import tpu_sc as plsc

B_ROWS = 256000
M = 128
K = 500
NSEQ = B_ROWS // K
HALF = K // 2
NGROUP = (HALF - 1) // 3
UNROLL = 4
NLOOP = NGROUP // UNROLL
NTAIL = NGROUP - NLOOP * UNROLL
NCHUNK = M // 16
L = 16
NW = 32
SEQ_PER_W = NSEQ // NW
SEQ_ELEMS = K * M
OUT_ELEMS = 6 * M


def _acc_rows(buf, accs, row0, nrows):
    accs = list(accs)
    for r in range(nrows):
        for c in range(NCHUNK):
            v = buf[pl.ds((row0 + r) * M + c * L, L)]
            k = (r % 3) * NCHUNK + c
            accs[k] = accs[k] + v
    return accs


def _sc_segment_mean(x):
    mesh = plsc.VectorSubcoreMesh(core_axis_name="c", subcore_axis_name="s")

    @functools.partial(
        pl.kernel,
        out_type=jax.ShapeDtypeStruct((NSEQ, OUT_ELEMS), jnp.float32),
        mesh=mesh,
        scratch_types=[
            pltpu.VMEM((HALF * M,), jnp.float32),
            pltpu.VMEM((HALF * M,), jnp.float32),
            pltpu.VMEM((HALF * M,), jnp.float32),
            pltpu.VMEM((SEQ_PER_W, OUT_ELEMS), jnp.float32),
            pltpu.SemaphoreType.DMA,
            pltpu.SemaphoreType.DMA,
            pltpu.SemaphoreType.DMA,
        ],
    )
    def body(x_hbm, out_hbm, buf0, buf1, buf2, obuf, sem0, sem1, sem2):
        wid = lax.axis_index("s") * 2 + lax.axis_index("c")
        base = wid * SEQ_PER_W
        bufs = (buf0, buf1, buf2)
        sems = (sem0, sem1, sem2)
        nhalf = 2 * SEQ_PER_W

        def start_dyn(j, k):
            src = x_hbm.at[pl.ds(base * SEQ_ELEMS + j * (HALF * M), HALF * M)]
            pltpu.async_copy(src, bufs[k], sems[k])

        def wait(k):
            pltpu.make_async_copy(
                x_hbm.at[pl.ds(0, HALF * M)], bufs[k], sems[k]
            ).wait()

        def compute(k, j):
            buf = bufs[k]
            i = j // 2
            h = j % 2

            def group_body(t, accs):
                return tuple(_acc_rows(buf, accs, 3 * UNROLL * t, 3 * UNROLL))

            init = tuple(jnp.zeros((L,), jnp.float32) for _ in range(3 * NCHUNK))
            accs = list(lax.fori_loop(0, NLOOP, group_body, init))
            accs = _acc_rows(buf, accs, 3 * NLOOP * UNROLL, 3 * NTAIL + 1)
            for r in range(3):
                scale = 1.0 / float(NGROUP + (1 if r == 0 else 0))
                for c in range(NCHUNK):
                    obuf[i, pl.ds((3 * h + r) * M + c * L, L)] = (
                        accs[r * NCHUNK + c] * scale
                    )

        start_dyn(0, 0)
        start_dyn(1, 1)

        def trio_body(t, carry):
            j0 = 3 * t
            start_dyn(j0 + 2, 2)
            wait(0)
            compute(0, j0)
            start_dyn(j0 + 3, 0)
            wait(1)
            compute(1, j0 + 1)
            start_dyn(j0 + 4, 1)
            wait(2)
            compute(2, j0 + 2)
            return carry

        lax.fori_loop(0, (nhalf - 2) // 3, trio_body, 0)
        wait(0)
        compute(0, nhalf - 2)
        wait(1)
        compute(1, nhalf - 1)
        pltpu.sync_copy(obuf, out_hbm.at[pl.ds(base, SEQ_PER_W)])

    return body(x)


def kernel(inputs):
    return _sc_segment_mean(inputs.reshape(-1))
